# Initial kernel scaffold; baseline (speedup 1.0000x reference)
import functools

import jax
import jax.numpy as jnp
from jax import lax
from jax.experimental import pallas as pl
from jax.experimental.pallas import tpu as pltpu
from jax.experimental.pallas import tpu_sc as plsc

_N = 10000
_E = 320000
_D = 128
_H = 64
_CH = 128
_NSUB = 16
_NCHUNKS = _E // _CH
_CPT = (_NCHUNKS + _NSUB - 1) // _NSUB
_ROWS_PER_TILE = _N // _NSUB

_HP = jax.lax.Precision.HIGHEST


def _hid_body(x_ref, wlt_ref, b_ref, h0_ref, h1_ref):
    acc = jnp.dot(x_ref[...], wlt_ref[...],
                  preferred_element_type=jnp.float32, precision=_HP)
    acc = acc + b_ref[...]
    h0_ref[...] = acc[:, :_H]
    h1_ref[...] = acc[:, _H:]


_hid_call = pl.pallas_call(
    _hid_body,
    grid=(5,),
    in_specs=[
        pl.BlockSpec((2000, _D), lambda i: (i, 0)),
        pl.BlockSpec((_D, _D), lambda i: (0, 0)),
        pl.BlockSpec((1, _D), lambda i: (0, 0)),
    ],
    out_specs=[
        pl.BlockSpec((2000, _H), lambda i: (i, 0)),
        pl.BlockSpec((2000, _H), lambda i: (i, 0)),
    ],
    out_shape=[jax.ShapeDtypeStruct((_N, _H), jnp.float32)] * 2,
)


_EB = 8000


def _edge_body(c_ref, w1_ref, b1_ref, w2t_ref, b2_ref, e0_ref, e1_ref):
    c = c_ref[...]
    h = c * w1_ref[...] + b1_ref[...]
    h = jnp.where(h >= 0, h, 0.2 * h)
    nc = jnp.dot(h, w2t_ref[...],
                 preferred_element_type=jnp.float32, precision=_HP)
    nc = nc + b2_ref[...]
    e = jnp.exp(nc)
    e0_ref[...] = e[:, :_H]
    e1_ref[...] = e[:, _H:]


_edge_call = pl.pallas_call(
    _edge_body,
    grid=(_E // _EB,),
    in_specs=[
        pl.BlockSpec((_EB, 1), lambda i: (i, 0)),
        pl.BlockSpec((1, _D), lambda i: (0, 0)),
        pl.BlockSpec((1, _D), lambda i: (0, 0)),
        pl.BlockSpec((_D, _D), lambda i: (0, 0)),
        pl.BlockSpec((1, _D), lambda i: (0, 0)),
    ],
    out_specs=[
        pl.BlockSpec((_EB, _H), lambda i: (i, 0)),
        pl.BlockSpec((_EB, _H), lambda i: (i, 0)),
    ],
    out_shape=[jax.ShapeDtypeStruct((_E, _H), jnp.float32)] * 2,
)


_mesh = plsc.VectorSubcoreMesh(core_axis_name="c", subcore_axis_name="s")


@functools.partial(
    pl.kernel,
    out_type=[jax.ShapeDtypeStruct((2, _N, _H), jnp.float32),
              jax.ShapeDtypeStruct((2, _N, _H), jnp.float32)],
    mesh=_mesh,
    scratch_types=[
        pltpu.VMEM((_CH,), jnp.int32),
        pltpu.VMEM((_CH,), jnp.int32),
        pltpu.VMEM((_CH, _H), jnp.float32),
        pltpu.VMEM((_CH, _H), jnp.float32),
        pltpu.VMEM((_CH, _H), jnp.float32),
        pltpu.VMEM_SHARED((_N, _H), jnp.float32),
        pltpu.VMEM_SHARED((_N, _H), jnp.float32),
        pltpu.SemaphoreType.DMA,
        pltpu.SemaphoreType.DMA,
    ],
)
def _sc_call(ei_hbm, ej_hbm, e0_hbm, e1_hbm, h0_hbm, h1_hbm, zero_hbm,
             u_hbm, s_hbm,
             idxg, idxs, hbuf, ebuf, pbuf, uacc, sacc, sem_e, sem_h):
    cid = lax.axis_index("c")
    sid = lax.axis_index("s")
    row0 = sid * _ROWS_PER_TILE

    pltpu.sync_copy(zero_hbm.at[pl.ds(row0, _ROWS_PER_TILE)],
                    uacc.at[pl.ds(row0, _ROWS_PER_TILE)])
    pltpu.sync_copy(zero_hbm.at[pl.ds(row0, _ROWS_PER_TILE)],
                    sacc.at[pl.ds(row0, _ROWS_PER_TILE)])
    plsc.subcore_barrier()

    def run(e_hbm, h_hbm):
        @pl.loop(0, _CPT)
        def _(it):
            k = it * _NSUB + sid

            @pl.when(k < _NCHUNKS)
            def _():
                base = k * _CH
                pltpu.sync_copy(ej_hbm.at[pl.ds(base, _CH)], idxg)
                pltpu.sync_copy(ei_hbm.at[pl.ds(base, _CH)], idxs)
                cp_e = pltpu.async_copy(e_hbm.at[pl.ds(base, _CH)], ebuf, sem_e)
                cp_h = pltpu.async_copy(h_hbm.at[idxg], hbuf, sem_h)
                cp_e.wait()
                cp_h.wait()

                @pl.loop(0, _CH)
                def _(r):
                    for q in range(_H // 16):
                        sl = pl.ds(q * 16, 16)
                        pbuf[r, sl] = hbuf[r, sl] * ebuf[r, sl]

                pltpu.sync_copy(ebuf, sacc.at[idxs], add=True)
                pltpu.sync_copy(pbuf, uacc.at[idxs], add=True)

    @pl.when(cid == 0)
    def _():
        run(e0_hbm, h0_hbm)

    @pl.when(cid == 1)
    def _():
        run(e1_hbm, h1_hbm)

    plsc.subcore_barrier()
    pltpu.sync_copy(uacc.at[pl.ds(row0, _ROWS_PER_TILE)],
                    u_hbm.at[cid].at[pl.ds(row0, _ROWS_PER_TILE)])
    pltpu.sync_copy(sacc.at[pl.ds(row0, _ROWS_PER_TILE)],
                    s_hbm.at[cid].at[pl.ds(row0, _ROWS_PER_TILE)])


_FB = 2000


def _fin_body(u_ref, s_ref, o_ref):
    u = u_ref[...]
    s = s_ref[...]
    r = u / (s + 1e-16)
    r = jnp.maximum(r, 0.0)
    o_ref[:, :_H] = r[0]
    o_ref[:, _H:] = r[1]


_fin_call = pl.pallas_call(
    _fin_body,
    grid=(_N // _FB,),
    in_specs=[
        pl.BlockSpec((2, _FB, _H), lambda i: (0, i, 0)),
        pl.BlockSpec((2, _FB, _H), lambda i: (0, i, 0)),
    ],
    out_specs=pl.BlockSpec((_FB, _D), lambda i: (i, 0)),
    out_shape=jax.ShapeDtypeStruct((_N, _D), jnp.float32),
)


def kernel(x, edge_index, curvature, W_lin, b_lin, W1, b1, W2, b2):
    ei = edge_index[0]
    ej = edge_index[1]
    h0, h1 = _hid_call(x, W_lin.T, b_lin.reshape(1, _D))
    e0, e1 = _edge_call(curvature, W1.reshape(1, _D), b1.reshape(1, _D),
                        W2.T, b2.reshape(1, _D))
    zeros = jnp.zeros((_N, _H), jnp.float32)
    u, s = _sc_call(ei, ej, e0, e1, h0, h1, zeros)
    return _fin_call(u, s)


# speedup vs baseline: 3.8893x; 3.8893x over previous
import functools

import jax
import jax.numpy as jnp
from jax import lax
from jax.experimental import pallas as pl
from jax.experimental.pallas import tpu as pltpu
from jax.experimental.pallas import tpu_sc as plsc

_N = 10000
_E = 320000
_D = 128
_H = 64
_CH = 128
_NSUB = 16
_NCHUNKS = _E // _CH
_CPT = (_NCHUNKS + _NSUB - 1) // _NSUB
_ROWS_A = 640
_ROWS_LAST = _N - _ROWS_A * (_NSUB - 1)

_HP = jax.lax.Precision.HIGHEST


def _hid_body(x_ref, wlt_ref, b_ref, h0_ref, h1_ref):
    acc = jnp.dot(x_ref[...], wlt_ref[...],
                  preferred_element_type=jnp.float32, precision=_HP)
    acc = acc + b_ref[...]
    h0_ref[...] = acc[:, :_H]
    h1_ref[...] = acc[:, _H:]


_hid_call = pl.pallas_call(
    _hid_body,
    grid=(5,),
    in_specs=[
        pl.BlockSpec((2000, _D), lambda i: (i, 0)),
        pl.BlockSpec((_D, _D), lambda i: (0, 0)),
        pl.BlockSpec((1, _D), lambda i: (0, 0)),
    ],
    out_specs=[
        pl.BlockSpec((2000, _H), lambda i: (i, 0)),
        pl.BlockSpec((2000, _H), lambda i: (i, 0)),
    ],
    out_shape=[jax.ShapeDtypeStruct((_N, _H), jnp.float32)] * 2,
)


_EB = 8000


def _edge_body(c_ref, w1_ref, b1_ref, w2t_ref, b2_ref, e0_ref, e1_ref):
    c = c_ref[...]
    h = c * w1_ref[...] + b1_ref[...]
    h = jnp.where(h >= 0, h, 0.2 * h)
    nc = jnp.dot(h, w2t_ref[...],
                 preferred_element_type=jnp.float32, precision=_HP)
    nc = nc + b2_ref[...]
    e = jnp.exp(nc)
    e0_ref[...] = e[:, :_H]
    e1_ref[...] = e[:, _H:]


_edge_call = pl.pallas_call(
    _edge_body,
    grid=(_E // _EB,),
    in_specs=[
        pl.BlockSpec((_EB, 1), lambda i: (i, 0)),
        pl.BlockSpec((1, _D), lambda i: (0, 0)),
        pl.BlockSpec((1, _D), lambda i: (0, 0)),
        pl.BlockSpec((_D, _D), lambda i: (0, 0)),
        pl.BlockSpec((1, _D), lambda i: (0, 0)),
    ],
    out_specs=[
        pl.BlockSpec((_EB, _H), lambda i: (i, 0)),
        pl.BlockSpec((_EB, _H), lambda i: (i, 0)),
    ],
    out_shape=[jax.ShapeDtypeStruct((_E, _H), jnp.float32)] * 2,
)


@functools.cache
def _make_sc_call():
    mesh = plsc.VectorSubcoreMesh(core_axis_name="c", subcore_axis_name="s")

    @functools.partial(
        pl.kernel,
        out_type=[jax.ShapeDtypeStruct((2, _N, _H), jnp.float32),
                  jax.ShapeDtypeStruct((2, _N, _H), jnp.float32)],
        mesh=mesh,
        compiler_params=pltpu.CompilerParams(use_tc_tiling_on_sc=False),
        scratch_types=[
            pltpu.VMEM((_CH,), jnp.int32),
            pltpu.VMEM((_CH,), jnp.int32),
            pltpu.VMEM((_CH, _H), jnp.float32),
            pltpu.VMEM((_CH, _H), jnp.float32),
            pltpu.VMEM((_CH, _H), jnp.float32),
            pltpu.VMEM_SHARED((_N, _H), jnp.float32),
            pltpu.VMEM_SHARED((_N, _H), jnp.float32),
            pltpu.SemaphoreType.DMA,
            pltpu.SemaphoreType.DMA,
        ],
    )
    def sc_call(ei_hbm, ej_hbm, e0_hbm, e1_hbm, h0_hbm, h1_hbm, zero_hbm,
                u_hbm, s_hbm,
                idxg, idxs, hbuf, ebuf, pbuf, uacc, sacc, sem_e, sem_h):
        cid = lax.axis_index("c")
        sid = lax.axis_index("s")
        row0 = sid * _ROWS_A

        @pl.when(sid < _NSUB - 1)
        def _():
            pltpu.sync_copy(zero_hbm.at[pl.ds(row0, _ROWS_A)],
                            uacc.at[pl.ds(row0, _ROWS_A)])
            pltpu.sync_copy(zero_hbm.at[pl.ds(row0, _ROWS_A)],
                            sacc.at[pl.ds(row0, _ROWS_A)])

        @pl.when(sid == _NSUB - 1)
        def _():
            pltpu.sync_copy(zero_hbm.at[pl.ds(row0, _ROWS_LAST)],
                            uacc.at[pl.ds(row0, _ROWS_LAST)])
            pltpu.sync_copy(zero_hbm.at[pl.ds(row0, _ROWS_LAST)],
                            sacc.at[pl.ds(row0, _ROWS_LAST)])

        plsc.subcore_barrier()

        def run(e_hbm, h_hbm):
            @pl.loop(0, _CPT)
            def _(it):
                k = it * _NSUB + sid

                @pl.when(k < _NCHUNKS)
                def _():
                    base = k * _CH
                    pltpu.sync_copy(ej_hbm.at[pl.ds(base, _CH)], idxg)
                    pltpu.sync_copy(ei_hbm.at[pl.ds(base, _CH)], idxs)
                    cp_e = pltpu.async_copy(e_hbm.at[pl.ds(base, _CH)], ebuf,
                                            sem_e)
                    cp_h = pltpu.async_copy(h_hbm.at[idxg], hbuf, sem_h)
                    cp_e.wait()
                    cp_h.wait()

                    @pl.loop(0, _CH)
                    def _(r):
                        for q in range(_H // 16):
                            sl = pl.ds(q * 16, 16)
                            pbuf[r, sl] = hbuf[r, sl] * ebuf[r, sl]

                    pltpu.sync_copy(ebuf, sacc.at[idxs], add=True)
                    pltpu.sync_copy(pbuf, uacc.at[idxs], add=True)

        @pl.when(cid == 0)
        def _():
            run(e0_hbm, h0_hbm)

        @pl.when(cid == 1)
        def _():
            run(e1_hbm, h1_hbm)

        plsc.subcore_barrier()

        @pl.when(sid < _NSUB - 1)
        def _():
            pltpu.sync_copy(uacc.at[pl.ds(row0, _ROWS_A)],
                            u_hbm.at[cid].at[pl.ds(row0, _ROWS_A)])
            pltpu.sync_copy(sacc.at[pl.ds(row0, _ROWS_A)],
                            s_hbm.at[cid].at[pl.ds(row0, _ROWS_A)])

        @pl.when(sid == _NSUB - 1)
        def _():
            pltpu.sync_copy(uacc.at[pl.ds(row0, _ROWS_LAST)],
                            u_hbm.at[cid].at[pl.ds(row0, _ROWS_LAST)])
            pltpu.sync_copy(sacc.at[pl.ds(row0, _ROWS_LAST)],
                            s_hbm.at[cid].at[pl.ds(row0, _ROWS_LAST)])

    return sc_call


_FB = 2000


def _fin_body(u_ref, s_ref, o_ref):
    u = u_ref[...]
    s = s_ref[...]
    r = u / (s + 1e-16)
    r = jnp.maximum(r, 0.0)
    o_ref[:, :_H] = r[0]
    o_ref[:, _H:] = r[1]


_fin_call = pl.pallas_call(
    _fin_body,
    grid=(_N // _FB,),
    in_specs=[
        pl.BlockSpec((2, _FB, _H), lambda i: (0, i, 0)),
        pl.BlockSpec((2, _FB, _H), lambda i: (0, i, 0)),
    ],
    out_specs=pl.BlockSpec((_FB, _D), lambda i: (i, 0)),
    out_shape=jax.ShapeDtypeStruct((_N, _D), jnp.float32),
)


def kernel(x, edge_index, curvature, W_lin, b_lin, W1, b1, W2, b2):
    ei = edge_index[0]
    ej = edge_index[1]
    h0, h1 = _hid_call(x, W_lin.T, b_lin.reshape(1, _D))
    e0, e1 = _edge_call(curvature, W1.reshape(1, _D), b1.reshape(1, _D),
                        W2.T, b2.reshape(1, _D))
    zeros = jnp.zeros((_N, _H), jnp.float32)
    u, s = _make_sc_call()(ei, ej, e0, e1, h0, h1, zeros)
    return _fin_call(u, s)


# speedup vs baseline: 5.2235x; 1.3430x over previous
import functools

import jax
import jax.numpy as jnp
from jax import lax
from jax.experimental import pallas as pl
from jax.experimental.pallas import tpu as pltpu
from jax.experimental.pallas import tpu_sc as plsc

_N = 10000
_E = 320000
_D = 128
_H = 64
_CH = 128
_NSUB = 16
_NCHUNKS = _E // _CH
_CPT = (_NCHUNKS + _NSUB - 1) // _NSUB
_ROWS_A = 640
_ROWS_LAST = _N - _ROWS_A * (_NSUB - 1)

_HP = jax.lax.Precision.HIGHEST


def _hid_body(x_ref, wlt_ref, b_ref, h0_ref, h1_ref):
    acc = jnp.dot(x_ref[...], wlt_ref[...],
                  preferred_element_type=jnp.float32, precision=_HP)
    acc = acc + b_ref[...]
    h0_ref[...] = acc[:, :_H]
    h1_ref[...] = acc[:, _H:]


_hid_call = pl.pallas_call(
    _hid_body,
    grid=(5,),
    in_specs=[
        pl.BlockSpec((2000, _D), lambda i: (i, 0)),
        pl.BlockSpec((_D, _D), lambda i: (0, 0)),
        pl.BlockSpec((1, _D), lambda i: (0, 0)),
    ],
    out_specs=[
        pl.BlockSpec((2000, _H), lambda i: (i, 0)),
        pl.BlockSpec((2000, _H), lambda i: (i, 0)),
    ],
    out_shape=[jax.ShapeDtypeStruct((_N, _H), jnp.float32)] * 2,
)


_EB = 8000


def _edge_body(c_ref, w1_ref, b1_ref, w2t_ref, b2_ref, e_ref):
    c = c_ref[...]
    h = c * w1_ref[...] + b1_ref[...]
    h = jnp.where(h >= 0, h, 0.2 * h)
    nc = jnp.dot(h, w2t_ref[...],
                 preferred_element_type=jnp.float32, precision=_HP)
    nc = nc + b2_ref[...]
    e_ref[...] = jnp.exp(nc)


_edge_call = pl.pallas_call(
    _edge_body,
    grid=(_E // _EB,),
    in_specs=[
        pl.BlockSpec((_EB, 1), lambda i: (i, 0)),
        pl.BlockSpec((1, _D), lambda i: (0, 0)),
        pl.BlockSpec((1, _D), lambda i: (0, 0)),
        pl.BlockSpec((_D, _D), lambda i: (0, 0)),
        pl.BlockSpec((1, _D), lambda i: (0, 0)),
    ],
    out_specs=pl.BlockSpec((_EB, _D), lambda i: (i, 0)),
    out_shape=jax.ShapeDtypeStruct((_E, _D), jnp.float32),
)


@functools.cache
def _make_sc_call():
    mesh = plsc.VectorSubcoreMesh(core_axis_name="c", subcore_axis_name="s")

    @functools.partial(
        pl.kernel,
        out_type=[jax.ShapeDtypeStruct((_N, _D), jnp.float32),
                  jax.ShapeDtypeStruct((_N, _D), jnp.float32)],
        mesh=mesh,
        compiler_params=pltpu.CompilerParams(use_tc_tiling_on_sc=False),
        scratch_types=[
            pltpu.VMEM((_CH,), jnp.int32),
            pltpu.VMEM((_CH,), jnp.int32),
            pltpu.VMEM((_CH, _H), jnp.float32),
            pltpu.VMEM((_CH, _H), jnp.float32),
            pltpu.VMEM((_CH, _H), jnp.float32),
            pltpu.VMEM_SHARED((_N, _H), jnp.float32),
            pltpu.VMEM_SHARED((_N, _H), jnp.float32),
            pltpu.SemaphoreType.DMA,
            pltpu.SemaphoreType.DMA,
        ],
    )
    def sc_call(ei_hbm, ej_hbm, e_hbm, h0_hbm, h1_hbm, zero_hbm,
                u_hbm, s_hbm,
                idxg, idxs, hbuf, ebuf, pbuf, uacc, sacc, sem_e, sem_h):
        cid = lax.axis_index("c")
        sid = lax.axis_index("s")
        row0 = sid * _ROWS_A
        col0 = cid * _H

        @pl.when(sid < _NSUB - 1)
        def _():
            pltpu.sync_copy(zero_hbm.at[pl.ds(row0, _ROWS_A)],
                            uacc.at[pl.ds(row0, _ROWS_A)])
            pltpu.sync_copy(zero_hbm.at[pl.ds(row0, _ROWS_A)],
                            sacc.at[pl.ds(row0, _ROWS_A)])

        @pl.when(sid == _NSUB - 1)
        def _():
            pltpu.sync_copy(zero_hbm.at[pl.ds(row0, _ROWS_LAST)],
                            uacc.at[pl.ds(row0, _ROWS_LAST)])
            pltpu.sync_copy(zero_hbm.at[pl.ds(row0, _ROWS_LAST)],
                            sacc.at[pl.ds(row0, _ROWS_LAST)])

        plsc.subcore_barrier()

        def run(h_hbm):
            @pl.loop(0, _CPT)
            def _(it):
                k = it * _NSUB + sid

                @pl.when(k < _NCHUNKS)
                def _():
                    base = k * _CH
                    pltpu.sync_copy(ej_hbm.at[pl.ds(base, _CH)], idxg)
                    pltpu.sync_copy(ei_hbm.at[pl.ds(base, _CH)], idxs)
                    cp_e = pltpu.async_copy(
                        e_hbm.at[pl.ds(base, _CH), pl.ds(col0, _H)], ebuf,
                        sem_e)
                    cp_h = pltpu.async_copy(h_hbm.at[idxg], hbuf, sem_h)
                    cp_e.wait()
                    cp_h.wait()

                    @pl.loop(0, _CH)
                    def _(r):
                        for q in range(_H // 16):
                            sl = pl.ds(q * 16, 16)
                            pbuf[r, sl] = hbuf[r, sl] * ebuf[r, sl]

                    pltpu.sync_copy(ebuf, sacc.at[idxs], add=True)
                    pltpu.sync_copy(pbuf, uacc.at[idxs], add=True)

        @pl.when(cid == 0)
        def _():
            run(h0_hbm)

        @pl.when(cid == 1)
        def _():
            run(h1_hbm)

        plsc.subcore_barrier()

        @pl.when(sid < _NSUB - 1)
        def _():
            pltpu.sync_copy(uacc.at[pl.ds(row0, _ROWS_A)],
                            u_hbm.at[pl.ds(row0, _ROWS_A), pl.ds(col0, _H)])
            pltpu.sync_copy(sacc.at[pl.ds(row0, _ROWS_A)],
                            s_hbm.at[pl.ds(row0, _ROWS_A), pl.ds(col0, _H)])

        @pl.when(sid == _NSUB - 1)
        def _():
            pltpu.sync_copy(uacc.at[pl.ds(row0, _ROWS_LAST)],
                            u_hbm.at[pl.ds(row0, _ROWS_LAST), pl.ds(col0, _H)])
            pltpu.sync_copy(sacc.at[pl.ds(row0, _ROWS_LAST)],
                            s_hbm.at[pl.ds(row0, _ROWS_LAST), pl.ds(col0, _H)])

    return sc_call


_FB = 2000


def _fin_body(u_ref, s_ref, o_ref):
    u = u_ref[...]
    s = s_ref[...]
    r = u / (s + 1e-16)
    o_ref[...] = jnp.maximum(r, 0.0)


_fin_call = pl.pallas_call(
    _fin_body,
    grid=(_N // _FB,),
    in_specs=[
        pl.BlockSpec((_FB, _D), lambda i: (i, 0)),
        pl.BlockSpec((_FB, _D), lambda i: (i, 0)),
    ],
    out_specs=pl.BlockSpec((_FB, _D), lambda i: (i, 0)),
    out_shape=jax.ShapeDtypeStruct((_N, _D), jnp.float32),
)


def kernel(x, edge_index, curvature, W_lin, b_lin, W1, b1, W2, b2):
    ei = edge_index[0]
    ej = edge_index[1]
    h0, h1 = _hid_call(x, W_lin.T, b_lin.reshape(1, _D))
    e = _edge_call(curvature, W1.reshape(1, _D), b1.reshape(1, _D),
                   W2.T, b2.reshape(1, _D))
    zeros = jnp.zeros((_N, _H), jnp.float32)
    u, s = _make_sc_call()(ei, ej, e, h0, h1, zeros)
    return _fin_call(u, s)


# speedup vs baseline: 6.5843x; 1.2605x over previous
import functools

import jax
import jax.numpy as jnp
from jax import lax
from jax.experimental import pallas as pl
from jax.experimental.pallas import tpu as pltpu
from jax.experimental.pallas import tpu_sc as plsc

_N = 10000
_E = 320000
_D = 128
_H = 64
_CH = 128
_NSUB = 16
_NCHUNKS = _E // _CH
_CPT = (_NCHUNKS + _NSUB - 1) // _NSUB
_NB = 2
_CPT2 = ((_CPT + _NB - 1) // _NB) * _NB
_ROWS_A = 640
_ROWS_LAST = _N - _ROWS_A * (_NSUB - 1)

_HP = jax.lax.Precision.HIGHEST


def _hid_body(x_ref, wlt_ref, b_ref, h0_ref, h1_ref):
    acc = jnp.dot(x_ref[...], wlt_ref[...],
                  preferred_element_type=jnp.float32, precision=_HP)
    acc = acc + b_ref[...]
    h0_ref[...] = acc[:, :_H]
    h1_ref[...] = acc[:, _H:]


_hid_call = pl.pallas_call(
    _hid_body,
    grid=(5,),
    in_specs=[
        pl.BlockSpec((2000, _D), lambda i: (i, 0)),
        pl.BlockSpec((_D, _D), lambda i: (0, 0)),
        pl.BlockSpec((1, _D), lambda i: (0, 0)),
    ],
    out_specs=[
        pl.BlockSpec((2000, _H), lambda i: (i, 0)),
        pl.BlockSpec((2000, _H), lambda i: (i, 0)),
    ],
    out_shape=[jax.ShapeDtypeStruct((_N, _H), jnp.float32)] * 2,
)


_EB = 8000


def _edge_body(c_ref, w1_ref, b1_ref, w2t_ref, b2_ref, e_ref):
    c = c_ref[...]
    h = c * w1_ref[...] + b1_ref[...]
    h = jnp.where(h >= 0, h, 0.2 * h)
    nc = jnp.dot(h, w2t_ref[...],
                 preferred_element_type=jnp.float32, precision=_HP)
    nc = nc + b2_ref[...]
    e_ref[...] = jnp.exp(nc)


_edge_call = pl.pallas_call(
    _edge_body,
    grid=(_E // _EB,),
    in_specs=[
        pl.BlockSpec((_EB, 1), lambda i: (i, 0)),
        pl.BlockSpec((1, _D), lambda i: (0, 0)),
        pl.BlockSpec((1, _D), lambda i: (0, 0)),
        pl.BlockSpec((_D, _D), lambda i: (0, 0)),
        pl.BlockSpec((1, _D), lambda i: (0, 0)),
    ],
    out_specs=pl.BlockSpec((_EB, _D), lambda i: (i, 0)),
    out_shape=jax.ShapeDtypeStruct((_E, _D), jnp.float32),
)


@functools.cache
def _make_sc_call():
    mesh = plsc.VectorSubcoreMesh(core_axis_name="c", subcore_axis_name="s")

    @functools.partial(
        pl.kernel,
        out_type=[jax.ShapeDtypeStruct((_N, _D), jnp.float32),
                  jax.ShapeDtypeStruct((_N, _D), jnp.float32)],
        mesh=mesh,
        compiler_params=pltpu.CompilerParams(use_tc_tiling_on_sc=False),
        scratch_types=[
            pltpu.VMEM((_CH,), jnp.int32),
            pltpu.VMEM((_CH,), jnp.int32),
            pltpu.VMEM((_CH,), jnp.int32),
            pltpu.VMEM((_CH,), jnp.int32),
            pltpu.VMEM((_CH, _H), jnp.float32),
            pltpu.VMEM((_CH, _H), jnp.float32),
            pltpu.VMEM((_CH, _H), jnp.float32),
            pltpu.VMEM((_CH, _H), jnp.float32),
            pltpu.VMEM((_CH, _H), jnp.float32),
            pltpu.VMEM_SHARED((_N, _H), jnp.float32),
            pltpu.VMEM_SHARED((_N, _H), jnp.float32),
            pltpu.SemaphoreType.DMA,
            pltpu.SemaphoreType.DMA,
            pltpu.SemaphoreType.DMA,
            pltpu.SemaphoreType.DMA,
        ],
    )
    def sc_call(ei_hbm, ej_hbm, e_hbm, h0_hbm, h1_hbm, zero_hbm,
                u_hbm, s_hbm,
                idxg0, idxg1, idxs0, idxs1, hbuf0, hbuf1, ebuf0, ebuf1,
                pbuf, uacc, sacc, sem_e0, sem_e1, sem_h0, sem_h1):
        idxg = (idxg0, idxg1)
        idxs = (idxs0, idxs1)
        hbuf = (hbuf0, hbuf1)
        ebuf = (ebuf0, ebuf1)
        sem_e = (sem_e0, sem_e1)
        sem_h = (sem_h0, sem_h1)
        cid = lax.axis_index("c")
        sid = lax.axis_index("s")
        row0 = sid * _ROWS_A
        col0 = cid * _H

        @pl.when(sid < _NSUB - 1)
        def _():
            pltpu.sync_copy(zero_hbm.at[pl.ds(row0, _ROWS_A)],
                            uacc.at[pl.ds(row0, _ROWS_A)])
            pltpu.sync_copy(zero_hbm.at[pl.ds(row0, _ROWS_A)],
                            sacc.at[pl.ds(row0, _ROWS_A)])

        @pl.when(sid == _NSUB - 1)
        def _():
            pltpu.sync_copy(zero_hbm.at[pl.ds(row0, _ROWS_LAST)],
                            uacc.at[pl.ds(row0, _ROWS_LAST)])
            pltpu.sync_copy(zero_hbm.at[pl.ds(row0, _ROWS_LAST)],
                            sacc.at[pl.ds(row0, _ROWS_LAST)])

        plsc.subcore_barrier()

        def run(h_hbm):
            def issue(it, b):
                k = it * _NSUB + sid

                @pl.when(k < _NCHUNKS)
                def _():
                    base = k * _CH
                    pltpu.sync_copy(ej_hbm.at[pl.ds(base, _CH)], idxg[b])
                    pltpu.sync_copy(ei_hbm.at[pl.ds(base, _CH)], idxs[b])
                    pltpu.async_copy(
                        e_hbm.at[pl.ds(base, _CH), pl.ds(col0, _H)], ebuf[b],
                        sem_e[b])
                    pltpu.async_copy(h_hbm.at[idxg[b]], hbuf[b], sem_h[b])

            def process(it, b):
                k = it * _NSUB + sid

                @pl.when(k < _NCHUNKS)
                def _():
                    pltpu.make_async_copy(
                        e_hbm.at[pl.ds(0, _CH), pl.ds(col0, _H)], ebuf[b],
                        sem_e[b]).wait()
                    pltpu.make_async_copy(
                        h_hbm.at[idxg[b]], hbuf[b], sem_h[b]).wait()

                    @pl.loop(0, _CH)
                    def _(r):
                        for q in range(_H // 16):
                            sl = pl.ds(q * 16, 16)
                            pbuf[r, sl] = hbuf[b][r, sl] * ebuf[b][r, sl]

                    pltpu.sync_copy(ebuf[b], sacc.at[idxs[b]], add=True)
                    pltpu.sync_copy(pbuf, uacc.at[idxs[b]], add=True)

            for b in range(_NB):
                issue(b, b)

            @pl.loop(0, _CPT2, step=_NB)
            def _(it):
                for b in range(_NB):
                    process(it + b, b)
                    issue(it + b + _NB, b)

        @pl.when(cid == 0)
        def _():
            run(h0_hbm)

        @pl.when(cid == 1)
        def _():
            run(h1_hbm)

        plsc.subcore_barrier()

        @pl.when(sid < _NSUB - 1)
        def _():
            pltpu.sync_copy(uacc.at[pl.ds(row0, _ROWS_A)],
                            u_hbm.at[pl.ds(row0, _ROWS_A), pl.ds(col0, _H)])
            pltpu.sync_copy(sacc.at[pl.ds(row0, _ROWS_A)],
                            s_hbm.at[pl.ds(row0, _ROWS_A), pl.ds(col0, _H)])

        @pl.when(sid == _NSUB - 1)
        def _():
            pltpu.sync_copy(uacc.at[pl.ds(row0, _ROWS_LAST)],
                            u_hbm.at[pl.ds(row0, _ROWS_LAST), pl.ds(col0, _H)])
            pltpu.sync_copy(sacc.at[pl.ds(row0, _ROWS_LAST)],
                            s_hbm.at[pl.ds(row0, _ROWS_LAST), pl.ds(col0, _H)])

    return sc_call


_FB = 2000


def _fin_body(u_ref, s_ref, o_ref):
    u = u_ref[...]
    s = s_ref[...]
    r = u / (s + 1e-16)
    o_ref[...] = jnp.maximum(r, 0.0)


_fin_call = pl.pallas_call(
    _fin_body,
    grid=(_N // _FB,),
    in_specs=[
        pl.BlockSpec((_FB, _D), lambda i: (i, 0)),
        pl.BlockSpec((_FB, _D), lambda i: (i, 0)),
    ],
    out_specs=pl.BlockSpec((_FB, _D), lambda i: (i, 0)),
    out_shape=jax.ShapeDtypeStruct((_N, _D), jnp.float32),
)


def kernel(x, edge_index, curvature, W_lin, b_lin, W1, b1, W2, b2):
    ei = edge_index[0]
    ej = edge_index[1]
    h0, h1 = _hid_call(x, W_lin.T, b_lin.reshape(1, _D))
    e = _edge_call(curvature, W1.reshape(1, _D), b1.reshape(1, _D),
                   W2.T, b2.reshape(1, _D))
    zeros = jnp.zeros((_N, _H), jnp.float32)
    u, s = _make_sc_call()(ei, ej, e, h0, h1, zeros)
    return _fin_call(u, s)


# speedup vs baseline: 7.3053x; 1.1095x over previous
import functools

import jax
import jax.numpy as jnp
from jax import lax
from jax.experimental import pallas as pl
from jax.experimental.pallas import tpu as pltpu
from jax.experimental.pallas import tpu_sc as plsc

_N = 10000
_E = 320000
_EH = _E // 2
_D = 128
_H = 64
_CH = 128
_NSUB = 16
_NCHUNKS = _EH // _CH
_CPT = (_NCHUNKS + _NSUB - 1) // _NSUB
_NB = 2
_CPT2 = ((_CPT + _NB - 1) // _NB) * _NB
_ROWS_A = 640
_ROWS_LAST = _N - _ROWS_A * (_NSUB - 1)

_HP = jax.lax.Precision.HIGHEST


def _hid_body(x_ref, wlt_ref, b_ref, h0_ref, h1_ref):
    acc = jnp.dot(x_ref[...], wlt_ref[...],
                  preferred_element_type=jnp.float32, precision=_HP)
    acc = acc + b_ref[...]
    h0_ref[...] = acc[:, :_H]
    h1_ref[...] = acc[:, _H:]


_hid_call = pl.pallas_call(
    _hid_body,
    grid=(5,),
    in_specs=[
        pl.BlockSpec((2000, _D), lambda i: (i, 0)),
        pl.BlockSpec((_D, _D), lambda i: (0, 0)),
        pl.BlockSpec((1, _D), lambda i: (0, 0)),
    ],
    out_specs=[
        pl.BlockSpec((2000, _H), lambda i: (i, 0)),
        pl.BlockSpec((2000, _H), lambda i: (i, 0)),
    ],
    out_shape=[jax.ShapeDtypeStruct((_N, _H), jnp.float32)] * 2,
)


_EB = 8000


def _edge_body(c_ref, w1_ref, b1_ref, w2t_ref, b2_ref, e_ref):
    c = c_ref[...]
    h = c * w1_ref[...] + b1_ref[...]
    h = jnp.where(h >= 0, h, 0.2 * h)
    nc = jnp.dot(h, w2t_ref[...],
                 preferred_element_type=jnp.float32, precision=_HP)
    nc = nc + b2_ref[...]
    e_ref[...] = jnp.exp(nc)


_edge_call = pl.pallas_call(
    _edge_body,
    grid=(_EH // _EB,),
    in_specs=[
        pl.BlockSpec((_EB, 1), lambda i: (i, 0)),
        pl.BlockSpec((1, _D), lambda i: (0, 0)),
        pl.BlockSpec((1, _D), lambda i: (0, 0)),
        pl.BlockSpec((_D, _D), lambda i: (0, 0)),
        pl.BlockSpec((1, _D), lambda i: (0, 0)),
    ],
    out_specs=pl.BlockSpec((_EB, _D), lambda i: (i, 0)),
    out_shape=jax.ShapeDtypeStruct((_EH, _D), jnp.float32),
)


@functools.cache
def _make_sc_call(eoff):
    mesh = plsc.VectorSubcoreMesh(core_axis_name="c", subcore_axis_name="s")

    @functools.partial(
        pl.kernel,
        out_type=[jax.ShapeDtypeStruct((_N, _D), jnp.float32),
                  jax.ShapeDtypeStruct((_N, _D), jnp.float32)],
        mesh=mesh,
        compiler_params=pltpu.CompilerParams(use_tc_tiling_on_sc=False),
        scratch_types=[
            pltpu.VMEM((_CH,), jnp.int32),
            pltpu.VMEM((_CH,), jnp.int32),
            pltpu.VMEM((_CH,), jnp.int32),
            pltpu.VMEM((_CH,), jnp.int32),
            pltpu.VMEM((_CH, _H), jnp.float32),
            pltpu.VMEM((_CH, _H), jnp.float32),
            pltpu.VMEM((_CH, _H), jnp.float32),
            pltpu.VMEM((_CH, _H), jnp.float32),
            pltpu.VMEM((_CH, _H), jnp.float32),
            pltpu.VMEM_SHARED((_N, _H), jnp.float32),
            pltpu.VMEM_SHARED((_N, _H), jnp.float32),
            pltpu.SemaphoreType.DMA,
            pltpu.SemaphoreType.DMA,
            pltpu.SemaphoreType.DMA,
            pltpu.SemaphoreType.DMA,
        ],
    )
    def sc_call(ei_hbm, ej_hbm, e_hbm, h0_hbm, h1_hbm, zero_hbm,
                u_hbm, s_hbm,
                idxg0, idxg1, idxs0, idxs1, hbuf0, hbuf1, ebuf0, ebuf1,
                pbuf, uacc, sacc, sem_e0, sem_e1, sem_h0, sem_h1):
        idxg = (idxg0, idxg1)
        idxs = (idxs0, idxs1)
        hbuf = (hbuf0, hbuf1)
        ebuf = (ebuf0, ebuf1)
        sem_e = (sem_e0, sem_e1)
        sem_h = (sem_h0, sem_h1)
        cid = lax.axis_index("c")
        sid = lax.axis_index("s")
        row0 = sid * _ROWS_A
        col0 = cid * _H

        @pl.when(sid < _NSUB - 1)
        def _():
            pltpu.sync_copy(zero_hbm.at[pl.ds(row0, _ROWS_A)],
                            uacc.at[pl.ds(row0, _ROWS_A)])
            pltpu.sync_copy(zero_hbm.at[pl.ds(row0, _ROWS_A)],
                            sacc.at[pl.ds(row0, _ROWS_A)])

        @pl.when(sid == _NSUB - 1)
        def _():
            pltpu.sync_copy(zero_hbm.at[pl.ds(row0, _ROWS_LAST)],
                            uacc.at[pl.ds(row0, _ROWS_LAST)])
            pltpu.sync_copy(zero_hbm.at[pl.ds(row0, _ROWS_LAST)],
                            sacc.at[pl.ds(row0, _ROWS_LAST)])

        plsc.subcore_barrier()

        def run(h_hbm):
            def issue(it, b):
                k = it * _NSUB + sid

                @pl.when(k < _NCHUNKS)
                def _():
                    base = k * _CH
                    pltpu.sync_copy(ej_hbm.at[pl.ds(eoff + base, _CH)],
                                    idxg[b])
                    pltpu.sync_copy(ei_hbm.at[pl.ds(eoff + base, _CH)],
                                    idxs[b])
                    pltpu.async_copy(
                        e_hbm.at[pl.ds(base, _CH), pl.ds(col0, _H)], ebuf[b],
                        sem_e[b])
                    pltpu.async_copy(h_hbm.at[idxg[b]], hbuf[b], sem_h[b])

            def process(it, b):
                k = it * _NSUB + sid

                @pl.when(k < _NCHUNKS)
                def _():
                    pltpu.make_async_copy(
                        e_hbm.at[pl.ds(0, _CH), pl.ds(col0, _H)], ebuf[b],
                        sem_e[b]).wait()
                    pltpu.make_async_copy(
                        h_hbm.at[idxg[b]], hbuf[b], sem_h[b]).wait()

                    @pl.loop(0, _CH)
                    def _(r):
                        for q in range(_H // 16):
                            sl = pl.ds(q * 16, 16)
                            pbuf[r, sl] = hbuf[b][r, sl] * ebuf[b][r, sl]

                    pltpu.sync_copy(ebuf[b], sacc.at[idxs[b]], add=True)
                    pltpu.sync_copy(pbuf, uacc.at[idxs[b]], add=True)

            for b in range(_NB):
                issue(b, b)

            @pl.loop(0, _CPT2, step=_NB)
            def _(it):
                for b in range(_NB):
                    process(it + b, b)
                    issue(it + b + _NB, b)

        @pl.when(cid == 0)
        def _():
            run(h0_hbm)

        @pl.when(cid == 1)
        def _():
            run(h1_hbm)

        plsc.subcore_barrier()

        @pl.when(sid < _NSUB - 1)
        def _():
            pltpu.sync_copy(uacc.at[pl.ds(row0, _ROWS_A)],
                            u_hbm.at[pl.ds(row0, _ROWS_A), pl.ds(col0, _H)])
            pltpu.sync_copy(sacc.at[pl.ds(row0, _ROWS_A)],
                            s_hbm.at[pl.ds(row0, _ROWS_A), pl.ds(col0, _H)])

        @pl.when(sid == _NSUB - 1)
        def _():
            pltpu.sync_copy(uacc.at[pl.ds(row0, _ROWS_LAST)],
                            u_hbm.at[pl.ds(row0, _ROWS_LAST), pl.ds(col0, _H)])
            pltpu.sync_copy(sacc.at[pl.ds(row0, _ROWS_LAST)],
                            s_hbm.at[pl.ds(row0, _ROWS_LAST), pl.ds(col0, _H)])

    return sc_call


_FB = 2000


def _fin_body(u0_ref, s0_ref, u1_ref, s1_ref, o_ref):
    u = u0_ref[...] + u1_ref[...]
    s = s0_ref[...] + s1_ref[...]
    r = u / (s + 1e-16)
    o_ref[...] = jnp.maximum(r, 0.0)


_fin_call = pl.pallas_call(
    _fin_body,
    grid=(_N // _FB,),
    in_specs=[pl.BlockSpec((_FB, _D), lambda i: (i, 0))] * 4,
    out_specs=pl.BlockSpec((_FB, _D), lambda i: (i, 0)),
    out_shape=jax.ShapeDtypeStruct((_N, _D), jnp.float32),
)


def kernel(x, edge_index, curvature, W_lin, b_lin, W1, b1, W2, b2):
    ei = edge_index[0]
    ej = edge_index[1]
    h0, h1 = _hid_call(x, W_lin.T, b_lin.reshape(1, _D))
    w1r = W1.reshape(1, _D)
    b1r = b1.reshape(1, _D)
    w2t = W2.T
    b2r = b2.reshape(1, _D)
    c0 = lax.slice(curvature, (0, 0), (_EH, 1))
    c1 = lax.slice(curvature, (_EH, 0), (_E, 1))
    e0 = _edge_call(c0, w1r, b1r, w2t, b2r)
    e1 = _edge_call(c1, w1r, b1r, w2t, b2r)
    zeros = jnp.zeros((_N, _H), jnp.float32)
    u0, s0 = _make_sc_call(0)(ei, ej, e0, h0, h1, zeros)
    u0, s0, ei2, ej2, e1b = lax.optimization_barrier((u0, s0, ei, ej, e1))
    u1, s1 = _make_sc_call(_EH)(ei2, ej2, e1b, h0, h1, zeros)
    return _fin_call(u0, s0, u1, s1)
